# baseline (speedup 1.0000x reference)
import functools

import jax
import jax.numpy as jnp
from jax import lax
from jax.experimental import pallas as pl
from jax.experimental.pallas import tpu as pltpu
from jax.experimental.pallas import tpu_sc as plsc

N, D, B, T = 100000, 128, 4096, 8
H1, H2 = 128, 64
S1, S2 = 5, 2
W_POS, W_NOPOS = 0.6, 0.4

NW = 32
CH = 128
G1_ROWS = S1 * T * B
A1_ROWS = S1 * T * B
H2_ROWS = S2 * A1_ROWS
G1_PW = G1_ROWS // NW
H2_PW = H2_ROWS // NW
H0_PW = B // NW


def _sc_gather(x, idx1, idx2, nodes):
    mesh = plsc.VectorSubcoreMesh(core_axis_name="c", subcore_axis_name="s")

    @functools.partial(
        pl.kernel,
        out_type=(
            jax.ShapeDtypeStruct((G1_ROWS, D), jnp.float32),
            jax.ShapeDtypeStruct((A1_ROWS, D), jnp.float32),
            jax.ShapeDtypeStruct((B, D), jnp.float32),
        ),
        mesh=mesh,
        scratch_types=[
            pltpu.VMEM((H2_PW,), jnp.int32),
            pltpu.VMEM((CH, D), jnp.float32),
            pltpu.VMEM((CH, D), jnp.float32),
            pltpu.VMEM((CH, D), jnp.float32),
            pltpu.SemaphoreType.DMA,
            pltpu.SemaphoreType.DMA,
        ],
    )
    def k(x_hbm, idx1_hbm, idx2_hbm, nodes_hbm, g1_hbm, a1_hbm, h0_hbm,
          idx_v, buf_a, buf_b, obuf, sem_a, sem_b):
        wid = lax.axis_index("s") * 2 + lax.axis_index("c")

        def start_gather(c, buf, sem):
            pltpu.async_copy(x_hbm.at[idx_v.at[pl.ds(c * CH, CH)]], buf, sem)

        def wait_gather(c, buf, sem):
            pltpu.make_async_copy(
                x_hbm.at[idx_v.at[pl.ds(c * CH, CH)]], buf, sem).wait()

        def copy_phase(idx_hbm, n_pw, out_hbm):
            base = wid * n_pw
            pltpu.sync_copy(idx_hbm.at[pl.ds(base, n_pw)],
                            idx_v.at[pl.ds(0, n_pw)])
            nch = n_pw // CH
            start_gather(0, buf_a, sem_a)

            def body(g, _):
                c0 = 2 * g

                @pl.when(c0 + 1 < nch)
                def _():
                    start_gather(c0 + 1, buf_b, sem_b)

                wait_gather(c0, buf_a, sem_a)
                pltpu.sync_copy(buf_a, out_hbm.at[pl.ds(base + c0 * CH, CH)])

                @pl.when(c0 + 1 < nch)
                def _():

                    @pl.when(c0 + 2 < nch)
                    def _():
                        start_gather(c0 + 2, buf_a, sem_a)

                    wait_gather(c0 + 1, buf_b, sem_b)
                    pltpu.sync_copy(
                        buf_b, out_hbm.at[pl.ds(base + (c0 + 1) * CH, CH)])

                return 0

            lax.fori_loop(0, (nch + 1) // 2, body, 0)

        def reduce_phase():
            base_in = wid * H2_PW
            base_out = wid * (H2_PW // 2)
            pltpu.sync_copy(idx2_hbm.at[pl.ds(base_in, H2_PW)],
                            idx_v.at[pl.ds(0, H2_PW)])
            nch = H2_PW // CH
            OCH = CH // 2

            def pair_add(buf):
                def body(r, _):
                    for j in range(D // 16):
                        sl = pl.ds(j * 16, 16)
                        obuf[r, sl] = buf[2 * r, sl] + buf[2 * r + 1, sl]
                    return 0

                lax.fori_loop(0, OCH, body, 0)

            start_gather(0, buf_a, sem_a)

            def body(g, _):
                c0 = 2 * g
                start_gather(c0 + 1, buf_b, sem_b)
                wait_gather(c0, buf_a, sem_a)
                pair_add(buf_a)
                pltpu.sync_copy(
                    obuf.at[pl.ds(0, OCH)],
                    a1_hbm.at[pl.ds(base_out + c0 * OCH, OCH)])

                @pl.when(c0 + 2 < nch)
                def _():
                    start_gather(c0 + 2, buf_a, sem_a)

                wait_gather(c0 + 1, buf_b, sem_b)
                pair_add(buf_b)
                pltpu.sync_copy(
                    obuf.at[pl.ds(0, OCH)],
                    a1_hbm.at[pl.ds(base_out + (c0 + 1) * OCH, OCH)])
                return 0

            lax.fori_loop(0, nch // 2, body, 0)

        copy_phase(idx1_hbm, G1_PW, g1_hbm)
        reduce_phase()
        copy_phase(nodes_hbm, H0_PW, h0_hbm)

    return k(x, idx1, idx2, nodes)


def _tc_dense(g1, a1, h0, w1c, w2c, awp, awn, pe, wout, bout):
    NB = 16
    BT = B // NB
    M = S1 * BT

    def body(g1r, a1r, h0r, w1cr, w2cr, awpr, awnr, per,
             woutr, boutr, outr, seq, sm):
        b = pl.program_id(0)
        t = pl.program_id(1)

        @pl.when(jnp.logical_and(b == 0, t == 0))
        def _():
            ri = lax.broadcasted_iota(jnp.int32, (BT, M), 0)
            cj = lax.broadcasted_iota(jnp.int32, (BT, M), 1)
            dj = cj - ri * S1
            hit = jnp.logical_and(dj >= 0, dj < S1)
            sm[...] = jnp.where(hit, 1.0 / S1, 0.0).astype(jnp.bfloat16)

        w1c_ = w1cr[...]
        w2c_ = w2cr[...]
        sm_ = sm[...]
        gb = g1r[0].astype(jnp.bfloat16)
        ab = (a1r[0] * 0.5).astype(jnp.bfloat16)
        z1 = jnp.maximum(
            jnp.dot(jnp.concatenate([gb, ab], axis=1), w1c_,
                    preferred_element_type=jnp.float32), 0.0)
        agg0 = jnp.dot(sm_, gb, preferred_element_type=jnp.float32)
        zin0 = jnp.concatenate([h0r[...], agg0], axis=1).astype(jnp.bfloat16)
        z0 = jnp.maximum(
            jnp.dot(zin0, w1c_, preferred_element_type=jnp.float32), 0.0)
        agg2 = jnp.dot(sm_, z1.astype(jnp.bfloat16),
                       preferred_element_type=jnp.float32)
        zin2 = jnp.concatenate([z0, agg2], axis=1).astype(jnp.bfloat16)
        z2 = jnp.maximum(
            jnp.dot(zin2, w2c_, preferred_element_type=jnp.float32), 0.0)
        seq[pl.ds(t, 1)] = z2[None]

        @pl.when(t == T - 1)
        def _():
            sq = seq[...]

            def attn(s_, w_):
                sc_ = jnp.sum(s_ * w_[None, None, :], axis=-1, keepdims=True)
                m = jnp.max(sc_, axis=0, keepdims=True)
                e = jnp.exp(sc_ - m)
                wt = e / jnp.sum(e, axis=0, keepdims=True)
                return jnp.sum(s_ * wt, axis=0)

            pe_ = per[...]
            awp_ = awpr[...]
            awn_ = awnr[...]
            emb0 = (attn(sq + pe_[:, None, :], awp_[0]) * W_POS
                    + attn(sq, awn_[0]) * W_NOPOS)
            sq1 = jnp.stack([sq[0], sq[2], sq[4], sq[6]])
            emb1 = (attn(sq1 + pe_[0:4][:, None, :], awp_[1]) * W_POS
                    + attn(sq1, awn_[1]) * W_NOPOS)
            stacked = (emb0 + emb1) * 0.5
            outr[...] = stacked @ woutr[...] + boutr[...]

    return pl.pallas_call(
        body,
        grid=(NB, T),
        in_specs=[
            pl.BlockSpec((1, M, D), lambda b, t: (t, b, 0)),
            pl.BlockSpec((1, M, D), lambda b, t: (t, b, 0)),
            pl.BlockSpec((BT, D), lambda b, t: (b, 0)),
            pl.BlockSpec((2 * D, H1), lambda b, t: (0, 0)),
            pl.BlockSpec((2 * H1, H2), lambda b, t: (0, 0)),
            pl.BlockSpec((2, H2), lambda b, t: (0, 0)),
            pl.BlockSpec((2, H2), lambda b, t: (0, 0)),
            pl.BlockSpec((T, H2), lambda b, t: (0, 0)),
            pl.BlockSpec((H2, D), lambda b, t: (0, 0)),
            pl.BlockSpec((1, D), lambda b, t: (0, 0)),
        ],
        out_specs=pl.BlockSpec((BT, D), lambda b, t: (b, 0)),
        out_shape=jax.ShapeDtypeStruct((B, D), jnp.float32),
        scratch_shapes=[
            pltpu.VMEM((T, BT, H2), jnp.float32),
            pltpu.VMEM((BT, M), jnp.bfloat16),
        ],
        compiler_params=pltpu.CompilerParams(
            dimension_semantics=("arbitrary", "arbitrary"),
        ),
    )(g1, a1, h0, w1c, w2c, awp, awn, pe, wout, bout)


def kernel(x, nodes, nbr1, nbr2, W1_self, W1_nbr, W2_self, W2_nbr,
           attn_w_pos, attn_b_pos, attn_w_nopos, attn_b_nopos, pe, Wout,
           bout):
    del attn_b_pos, attn_b_nopos
    nodes_i = nodes.astype(jnp.int32).reshape(B)
    idx1 = nbr1.astype(jnp.int32).reshape(G1_ROWS)
    idx2 = nbr2.astype(jnp.int32).reshape(H2_ROWS)
    g1, a1, h0 = _sc_gather(x, idx1, idx2, nodes_i)
    w1c = jnp.concatenate([W1_self, W1_nbr], axis=0).astype(jnp.bfloat16)
    w2c = jnp.concatenate([W2_self, W2_nbr], axis=0).astype(jnp.bfloat16)
    return _tc_dense(
        g1.reshape(T, S1 * B, D), a1.reshape(T, S1 * B, D), h0,
        w1c, w2c, attn_w_pos, attn_w_nopos, pe, Wout, bout.reshape(1, D))


# speedup vs baseline: 1.4901x; 1.4901x over previous
import functools

import jax
import jax.numpy as jnp
from jax import lax
from jax.experimental import pallas as pl
from jax.experimental.pallas import tpu as pltpu
from jax.experimental.pallas import tpu_sc as plsc

N, D, B, T = 100000, 128, 4096, 8
H1, H2 = 128, 64
S1, S2 = 5, 2
W_POS, W_NOPOS = 0.6, 0.4

NW = 32
CH = 128
G1_ROWS = S1 * T * B
A1_ROWS = S1 * T * B
H2_ROWS = S2 * A1_ROWS
NP = S1 * T
H0_PW = B // NW


def _sc_gather(x, idx1, idx2, nodes):
    mesh = plsc.VectorSubcoreMesh(core_axis_name="c", subcore_axis_name="s")

    @functools.partial(
        pl.kernel,
        out_type=(
            jax.ShapeDtypeStruct((G1_ROWS, D), jnp.float32),
            jax.ShapeDtypeStruct((A1_ROWS, D), jnp.float32),
            jax.ShapeDtypeStruct((B, D), jnp.float32),
        ),
        mesh=mesh,
        scratch_types=[
            pltpu.VMEM((NP * CH * S2,), jnp.int32),
            pltpu.VMEM((NP * CH,), jnp.int32),
            pltpu.VMEM((NP * CH,), jnp.int32),
            pltpu.VMEM((CH, D), jnp.float32),
            pltpu.VMEM((CH, D), jnp.float32),
            pltpu.VMEM((CH, D), jnp.float32),
            pltpu.VMEM((CH, D), jnp.float32),
            pltpu.VMEM((CH, D), jnp.float32),
            pltpu.SemaphoreType.DMA,
            pltpu.SemaphoreType.DMA,
            pltpu.SemaphoreType.DMA,
            pltpu.SemaphoreType.DMA,
        ],
        compiler_params=pltpu.CompilerParams(needs_layout_passes=False),
    )
    def k(x_hbm, idx1_hbm, idx2_hbm, nodes_hbm, g1_hbm, a1_hbm, h0_hbm,
          iraw, ipa, ipb, buf_a, buf_b, buf_c, buf_d, obuf,
          sem_a, sem_b, sem_c, sem_d):
        wid = lax.axis_index("s") * 2 + lax.axis_index("c")
        b0 = wid * CH
        iota16 = lax.iota(jnp.int32, 16)

        def st_of(c):
            return lax.shift_right_logical(c, 3), lax.bitwise_and(c, 7)

        def load_deint(idx_hbm, width, outs):
            nsub = len(outs)
            for t in range(T):
                pltpu.sync_copy(
                    idx_hbm.at[pl.ds(t * B * width + b0 * width,
                                     CH * width)],
                    iraw.at[pl.ds(t * CH * width, CH * width)])

            def dbody(c, _):
                s, t = st_of(c)
                rbase = t * CH * width + s * nsub
                for kk in range(CH // 16):
                    for u, outv in enumerate(outs):
                        vals = plsc.load_gather(
                            iraw, [iota16 * width
                                   + (rbase + u + kk * 16 * width)])
                        outv[pl.ds(c * CH + kk * 16, 16)] = vals
                return 0

            lax.fori_loop(0, NP, dbody, 0)

        def start_gather(iv, off, buf, sem):
            pltpu.async_copy(
                x_hbm.at[iv.at[pl.ds(off, CH)]], buf, sem)

        def wait_gather(iv, off, buf, sem):
            pltpu.make_async_copy(
                x_hbm.at[iv.at[pl.ds(off, CH)]], buf, sem).wait()

        def g1_phase():
            start_gather(ipa, 0, buf_a, sem_a)

            def body(g, _):
                c0 = 2 * g
                start_gather(ipa, (c0 + 1) * CH, buf_b, sem_b)
                wait_gather(ipa, c0 * CH, buf_a, sem_a)
                pltpu.sync_copy(buf_a, g1_hbm.at[pl.ds(c0 * B + b0, CH)])

                @pl.when(c0 + 2 < NP)
                def _():
                    start_gather(ipa, (c0 + 2) * CH, buf_a, sem_a)

                wait_gather(ipa, (c0 + 1) * CH, buf_b, sem_b)
                pltpu.sync_copy(
                    buf_b, g1_hbm.at[pl.ds((c0 + 1) * B + b0, CH)])
                return 0

            lax.fori_loop(0, NP // 2, body, 0)

        def pair_add(bufx, bufy):
            def body(r, _):
                for j in range(D // 16):
                    sl = pl.ds(j * 16, 16)
                    obuf[r, sl] = bufx[r, sl] + bufy[r, sl]
                return 0

            lax.fori_loop(0, CH, body, 0)

        def a1_start(c, bufx, bufy, semx, semy):
            start_gather(ipa, c * CH, bufx, semx)
            start_gather(ipb, c * CH, bufy, semy)

        def a1_wait(c, bufx, bufy, semx, semy):
            wait_gather(ipa, c * CH, bufx, semx)
            wait_gather(ipb, c * CH, bufy, semy)

        def a1_phase():
            a1_start(0, buf_a, buf_b, sem_a, sem_b)

            def body(g, _):
                c0 = 2 * g
                a1_start(c0 + 1, buf_c, buf_d, sem_c, sem_d)
                a1_wait(c0, buf_a, buf_b, sem_a, sem_b)
                pair_add(buf_a, buf_b)
                pltpu.sync_copy(obuf, a1_hbm.at[pl.ds(c0 * B + b0, CH)])

                @pl.when(c0 + 2 < NP)
                def _():
                    a1_start(c0 + 2, buf_a, buf_b, sem_a, sem_b)

                a1_wait(c0 + 1, buf_c, buf_d, sem_c, sem_d)
                pair_add(buf_c, buf_d)
                pltpu.sync_copy(obuf,
                                a1_hbm.at[pl.ds((c0 + 1) * B + b0, CH)])
                return 0

            lax.fori_loop(0, NP // 2, body, 0)

        def h0_phase():
            pltpu.sync_copy(nodes_hbm.at[pl.ds(b0, CH)],
                            ipa.at[pl.ds(0, CH)])
            start_gather(ipa, 0, buf_a, sem_a)
            wait_gather(ipa, 0, buf_a, sem_a)
            pltpu.sync_copy(buf_a, h0_hbm.at[pl.ds(b0, CH)])

        load_deint(idx1_hbm, S1, [ipa])
        g1_phase()
        load_deint(idx2_hbm, S1 * S2, [ipa, ipb])
        a1_phase()
        h0_phase()

    return k(x, idx1, idx2, nodes)


def _tc_dense(g1, a1, h0, w1c, w2c, awp, awn, pe, wout, bout):
    NB = 16
    BT = B // NB

    def body(g1r, a1r, h0r, w1cr, w2cr, awpr, awnr, per,
             woutr, boutr, outr, seq):
        t = pl.program_id(1)
        w1c_ = w1cr[...]
        w2c_ = w2cr[...]
        g1f = g1r[...]
        a1f = a1r[...]
        gb = g1f.reshape(S1 * BT, D).astype(jnp.bfloat16)
        ab = (a1f.reshape(S1 * BT, D) * 0.5).astype(jnp.bfloat16)
        z1 = jnp.maximum(
            jnp.dot(jnp.concatenate([gb, ab], axis=1), w1c_,
                    preferred_element_type=jnp.float32), 0.0)
        agg0 = (g1f[0, 0] + g1f[1, 0] + g1f[2, 0] + g1f[3, 0]
                + g1f[4, 0]) * (1.0 / S1)
        zin0 = jnp.concatenate([h0r[...], agg0], axis=1).astype(jnp.bfloat16)
        z0 = jnp.maximum(
            jnp.dot(zin0, w1c_, preferred_element_type=jnp.float32), 0.0)
        agg2 = (z1[0:BT] + z1[BT:2 * BT] + z1[2 * BT:3 * BT]
                + z1[3 * BT:4 * BT] + z1[4 * BT:5 * BT]) * (1.0 / S1)
        zin2 = jnp.concatenate([z0, agg2], axis=1).astype(jnp.bfloat16)
        z2 = jnp.maximum(
            jnp.dot(zin2, w2c_, preferred_element_type=jnp.float32), 0.0)
        seq[pl.ds(t, 1)] = z2[None]

        @pl.when(t == T - 1)
        def _():
            sq = seq[...]

            def attn(s_, w_):
                sc_ = jnp.sum(s_ * w_[None, None, :], axis=-1, keepdims=True)
                m = jnp.max(sc_, axis=0, keepdims=True)
                e = jnp.exp(sc_ - m)
                wt = e / jnp.sum(e, axis=0, keepdims=True)
                return jnp.sum(s_ * wt, axis=0)

            pe_ = per[...]
            awp_ = awpr[...]
            awn_ = awnr[...]
            emb0 = (attn(sq + pe_[:, None, :], awp_[0]) * W_POS
                    + attn(sq, awn_[0]) * W_NOPOS)
            sq1 = jnp.stack([sq[0], sq[2], sq[4], sq[6]])
            emb1 = (attn(sq1 + pe_[0:4][:, None, :], awp_[1]) * W_POS
                    + attn(sq1, awn_[1]) * W_NOPOS)
            stacked = (emb0 + emb1) * 0.5
            outr[...] = stacked @ woutr[...] + boutr[...]

    return pl.pallas_call(
        body,
        grid=(NB, T),
        in_specs=[
            pl.BlockSpec((S1, 1, BT, D), lambda b, t: (0, t, b, 0)),
            pl.BlockSpec((S1, 1, BT, D), lambda b, t: (0, t, b, 0)),
            pl.BlockSpec((BT, D), lambda b, t: (b, 0)),
            pl.BlockSpec((2 * D, H1), lambda b, t: (0, 0)),
            pl.BlockSpec((2 * H1, H2), lambda b, t: (0, 0)),
            pl.BlockSpec((2, H2), lambda b, t: (0, 0)),
            pl.BlockSpec((2, H2), lambda b, t: (0, 0)),
            pl.BlockSpec((T, H2), lambda b, t: (0, 0)),
            pl.BlockSpec((H2, D), lambda b, t: (0, 0)),
            pl.BlockSpec((1, D), lambda b, t: (0, 0)),
        ],
        out_specs=pl.BlockSpec((BT, D), lambda b, t: (b, 0)),
        out_shape=jax.ShapeDtypeStruct((B, D), jnp.float32),
        scratch_shapes=[pltpu.VMEM((T, BT, H2), jnp.float32)],
        compiler_params=pltpu.CompilerParams(
            dimension_semantics=("arbitrary", "arbitrary"),
        ),
    )(g1, a1, h0, w1c, w2c, awp, awn, pe, wout, bout)


def kernel(x, nodes, nbr1, nbr2, W1_self, W1_nbr, W2_self, W2_nbr,
           attn_w_pos, attn_b_pos, attn_w_nopos, attn_b_nopos, pe, Wout,
           bout):
    del attn_b_pos, attn_b_nopos
    nodes_i = nodes.astype(jnp.int32).reshape(B)
    idx1 = nbr1.astype(jnp.int32).reshape(G1_ROWS)
    idx2 = nbr2.astype(jnp.int32).reshape(H2_ROWS)
    g1, a1, h0 = _sc_gather(x, idx1, idx2, nodes_i)
    w1c = jnp.concatenate([W1_self, W1_nbr], axis=0).astype(jnp.bfloat16)
    w2c = jnp.concatenate([W2_self, W2_nbr], axis=0).astype(jnp.bfloat16)
    return _tc_dense(
        g1.reshape(S1, T, B, D), a1.reshape(S1, T, B, D), h0,
        w1c, w2c, attn_w_pos, attn_w_nopos, pe, Wout, bout.reshape(1, D))


# speedup vs baseline: 1.6394x; 1.1002x over previous
import functools

import jax
import jax.numpy as jnp
from jax import lax
from jax.experimental import pallas as pl
from jax.experimental.pallas import tpu as pltpu
from jax.experimental.pallas import tpu_sc as plsc

N, D, B, T = 100000, 128, 4096, 8
H1, H2 = 128, 64
S1, S2 = 5, 2
W_POS, W_NOPOS = 0.6, 0.4

NW = 32
CH = 128
G1_ROWS = S1 * T * B
A1_ROWS = S1 * T * B
H2_ROWS = S2 * A1_ROWS
NP = S1 * T
TH = T // 2
H0_PW = B // NW


def _sc_gather(x, idx1, idx2, nodes, thalf):
    mesh = plsc.VectorSubcoreMesh(core_axis_name="c", subcore_axis_name="s")
    NPH = S1 * TH
    HROWS = S1 * TH * B
    out_type = [
        jax.ShapeDtypeStruct((HROWS, D), jnp.float32),
        jax.ShapeDtypeStruct((HROWS, D), jnp.float32),
    ]
    if thalf == 0:
        out_type.append(jax.ShapeDtypeStruct((B, D), jnp.float32))

    @functools.partial(
        pl.kernel,
        out_type=tuple(out_type),
        mesh=mesh,
        scratch_types=[
            pltpu.VMEM((NPH * CH * S2,), jnp.int32),
            pltpu.VMEM((NPH * CH,), jnp.int32),
            pltpu.VMEM((NPH * CH,), jnp.int32),
            pltpu.VMEM((CH, D), jnp.float32),
            pltpu.VMEM((CH, D), jnp.float32),
            pltpu.VMEM((CH, D), jnp.float32),
            pltpu.VMEM((CH, D), jnp.float32),
            pltpu.VMEM((CH, D), jnp.float32),
            pltpu.SemaphoreType.DMA,
            pltpu.SemaphoreType.DMA,
            pltpu.SemaphoreType.DMA,
            pltpu.SemaphoreType.DMA,
        ],
        compiler_params=pltpu.CompilerParams(needs_layout_passes=False),
    )
    def k(x_hbm, idx1_hbm, idx2_hbm, nodes_hbm, *rest):
        if thalf == 0:
            g1_hbm, a1_hbm, h0_hbm = rest[:3]
            rest = rest[3:]
        else:
            g1_hbm, a1_hbm = rest[:2]
            h0_hbm = None
            rest = rest[2:]
        (iraw, ipa, ipb, buf_a, buf_b, buf_c, buf_d, obuf,
         sem_a, sem_b, sem_c, sem_d) = rest
        wid = lax.axis_index("s") * 2 + lax.axis_index("c")
        b0 = wid * CH
        iota16 = lax.iota(jnp.int32, 16)

        def st_of(c):
            return lax.shift_right_logical(c, 2), lax.bitwise_and(c, 3)

        def load_deint(idx_hbm, width, outs):
            nsub = len(outs)
            for t in range(TH):
                tg = thalf * TH + t
                pltpu.sync_copy(
                    idx_hbm.at[pl.ds(tg * B * width + b0 * width,
                                     CH * width)],
                    iraw.at[pl.ds(t * CH * width, CH * width)])

            def dbody(c, _):
                s, t = st_of(c)
                rbase = t * CH * width + s * nsub
                for kk in range(CH // 16):
                    for u, outv in enumerate(outs):
                        vals = plsc.load_gather(
                            iraw, [iota16 * width
                                   + (rbase + u + kk * 16 * width)])
                        outv[pl.ds(c * CH + kk * 16, 16)] = vals
                return 0

            lax.fori_loop(0, NPH, dbody, 0)

        def start_gather(iv, off, buf, sem):
            pltpu.async_copy(
                x_hbm.at[iv.at[pl.ds(off, CH)]], buf, sem)

        def wait_gather(iv, off, buf, sem):
            pltpu.make_async_copy(
                x_hbm.at[iv.at[pl.ds(off, CH)]], buf, sem).wait()

        def g1_phase():
            start_gather(ipa, 0, buf_a, sem_a)

            def body(g, _):
                c0 = 2 * g
                start_gather(ipa, (c0 + 1) * CH, buf_b, sem_b)
                wait_gather(ipa, c0 * CH, buf_a, sem_a)
                pltpu.sync_copy(buf_a, g1_hbm.at[pl.ds(c0 * B + b0, CH)])

                @pl.when(c0 + 2 < NPH)
                def _():
                    start_gather(ipa, (c0 + 2) * CH, buf_a, sem_a)

                wait_gather(ipa, (c0 + 1) * CH, buf_b, sem_b)
                pltpu.sync_copy(
                    buf_b, g1_hbm.at[pl.ds((c0 + 1) * B + b0, CH)])
                return 0

            lax.fori_loop(0, NPH // 2, body, 0)

        def pair_add(bufx, bufy):
            def body(r, _):
                for j in range(D // 16):
                    sl = pl.ds(j * 16, 16)
                    obuf[r, sl] = bufx[r, sl] + bufy[r, sl]
                return 0

            lax.fori_loop(0, CH, body, 0)

        def a1_start(c, bufx, bufy, semx, semy):
            start_gather(ipa, c * CH, bufx, semx)
            start_gather(ipb, c * CH, bufy, semy)

        def a1_wait(c, bufx, bufy, semx, semy):
            wait_gather(ipa, c * CH, bufx, semx)
            wait_gather(ipb, c * CH, bufy, semy)

        def a1_phase():
            a1_start(0, buf_a, buf_b, sem_a, sem_b)

            def body(g, _):
                c0 = 2 * g
                a1_start(c0 + 1, buf_c, buf_d, sem_c, sem_d)
                a1_wait(c0, buf_a, buf_b, sem_a, sem_b)
                pair_add(buf_a, buf_b)
                pltpu.sync_copy(obuf, a1_hbm.at[pl.ds(c0 * B + b0, CH)])

                @pl.when(c0 + 2 < NPH)
                def _():
                    a1_start(c0 + 2, buf_a, buf_b, sem_a, sem_b)

                a1_wait(c0 + 1, buf_c, buf_d, sem_c, sem_d)
                pair_add(buf_c, buf_d)
                pltpu.sync_copy(obuf,
                                a1_hbm.at[pl.ds((c0 + 1) * B + b0, CH)])
                return 0

            lax.fori_loop(0, NPH // 2, body, 0)

        def h0_phase():
            pltpu.sync_copy(nodes_hbm.at[pl.ds(b0, CH)],
                            ipa.at[pl.ds(0, CH)])
            start_gather(ipa, 0, buf_a, sem_a)
            wait_gather(ipa, 0, buf_a, sem_a)
            pltpu.sync_copy(buf_a, h0_hbm.at[pl.ds(b0, CH)])

        load_deint(idx1_hbm, S1, [ipa])
        g1_phase()
        load_deint(idx2_hbm, S1 * S2, [ipa, ipb])
        a1_phase()
        if thalf == 0:
            h0_phase()

    return k(x, idx1, idx2, nodes)


def _tc_half1(g1, a1, h0, w1c, w2c):
    NB = 16
    BT = B // NB

    def wrapped(g1r, a1r, h0r, w1cr, w2cr, seqr):
        z2 = _dense_z2(g1r, a1r, h0r, w1cr, w2cr, BT)
        seqr[0] = z2

    return pl.pallas_call(
        wrapped,
        grid=(NB, TH),
        in_specs=[
            pl.BlockSpec((S1, 1, BT, D), lambda b, t: (0, t, b, 0)),
            pl.BlockSpec((S1, 1, BT, D), lambda b, t: (0, t, b, 0)),
            pl.BlockSpec((BT, D), lambda b, t: (b, 0)),
            pl.BlockSpec((2 * D, H1), lambda b, t: (0, 0)),
            pl.BlockSpec((2 * H1, H2), lambda b, t: (0, 0)),
        ],
        out_specs=pl.BlockSpec((1, BT, H2), lambda b, t: (t, b, 0)),
        out_shape=jax.ShapeDtypeStruct((TH, B, H2), jnp.float32),
        compiler_params=pltpu.CompilerParams(
            dimension_semantics=("arbitrary", "arbitrary"),
        ),
    )(g1, a1, h0, w1c, w2c)


def _dense_z2(g1r, a1r, h0r, w1cr, w2cr, BT):
    w1c_ = w1cr[...]
    w2c_ = w2cr[...]
    g1f = g1r[...]
    a1f = a1r[...]
    gb = g1f.reshape(S1 * BT, D).astype(jnp.bfloat16)
    ab = (a1f.reshape(S1 * BT, D) * 0.5).astype(jnp.bfloat16)
    z1 = jnp.maximum(
        jnp.dot(jnp.concatenate([gb, ab], axis=1), w1c_,
                preferred_element_type=jnp.float32), 0.0)
    agg0 = (g1f[0, 0] + g1f[1, 0] + g1f[2, 0] + g1f[3, 0]
            + g1f[4, 0]) * (1.0 / S1)
    zin0 = jnp.concatenate([h0r[...], agg0], axis=1).astype(jnp.bfloat16)
    z0 = jnp.maximum(
        jnp.dot(zin0, w1c_, preferred_element_type=jnp.float32), 0.0)
    agg2 = (z1[0:BT] + z1[BT:2 * BT] + z1[2 * BT:3 * BT]
            + z1[3 * BT:4 * BT] + z1[4 * BT:5 * BT]) * (1.0 / S1)
    zin2 = jnp.concatenate([z0, agg2], axis=1).astype(jnp.bfloat16)
    return jnp.maximum(
        jnp.dot(zin2, w2c_, preferred_element_type=jnp.float32), 0.0)


def _tc_half2(g1, a1, h0, seq_a, w1c, w2c, awp, awn, pe, wout, bout):
    NB = 16
    BT = B // NB

    def body(g1r, a1r, h0r, seqar, w1cr, w2cr, awpr, awnr, per,
             woutr, boutr, outr, seqb):
        t = pl.program_id(1)
        z2 = _dense_z2(g1r, a1r, h0r, w1cr, w2cr, BT)
        seqb[pl.ds(t, 1)] = z2[None]

        @pl.when(t == TH - 1)
        def _():
            sq = jnp.concatenate([seqar[...], seqb[...]], axis=0)

            def attn(s_, w_):
                sc_ = jnp.sum(s_ * w_[None, None, :], axis=-1, keepdims=True)
                m = jnp.max(sc_, axis=0, keepdims=True)
                e = jnp.exp(sc_ - m)
                wt = e / jnp.sum(e, axis=0, keepdims=True)
                return jnp.sum(s_ * wt, axis=0)

            pe_ = per[...]
            awp_ = awpr[...]
            awn_ = awnr[...]
            emb0 = (attn(sq + pe_[:, None, :], awp_[0]) * W_POS
                    + attn(sq, awn_[0]) * W_NOPOS)
            sq1 = jnp.stack([sq[0], sq[2], sq[4], sq[6]])
            emb1 = (attn(sq1 + pe_[0:4][:, None, :], awp_[1]) * W_POS
                    + attn(sq1, awn_[1]) * W_NOPOS)
            stacked = (emb0 + emb1) * 0.5
            outr[...] = stacked @ woutr[...] + boutr[...]

    return pl.pallas_call(
        body,
        grid=(NB, TH),
        in_specs=[
            pl.BlockSpec((S1, 1, BT, D), lambda b, t: (0, t, b, 0)),
            pl.BlockSpec((S1, 1, BT, D), lambda b, t: (0, t, b, 0)),
            pl.BlockSpec((BT, D), lambda b, t: (b, 0)),
            pl.BlockSpec((TH, BT, H2), lambda b, t: (0, b, 0)),
            pl.BlockSpec((2 * D, H1), lambda b, t: (0, 0)),
            pl.BlockSpec((2 * H1, H2), lambda b, t: (0, 0)),
            pl.BlockSpec((2, H2), lambda b, t: (0, 0)),
            pl.BlockSpec((2, H2), lambda b, t: (0, 0)),
            pl.BlockSpec((T, H2), lambda b, t: (0, 0)),
            pl.BlockSpec((H2, D), lambda b, t: (0, 0)),
            pl.BlockSpec((1, D), lambda b, t: (0, 0)),
        ],
        out_specs=pl.BlockSpec((BT, D), lambda b, t: (b, 0)),
        out_shape=jax.ShapeDtypeStruct((B, D), jnp.float32),
        scratch_shapes=[pltpu.VMEM((TH, BT, H2), jnp.float32)],
        compiler_params=pltpu.CompilerParams(
            dimension_semantics=("arbitrary", "arbitrary"),
        ),
    )(g1, a1, h0, seq_a, w1c, w2c, awp, awn, pe, wout, bout)


def kernel(x, nodes, nbr1, nbr2, W1_self, W1_nbr, W2_self, W2_nbr,
           attn_w_pos, attn_b_pos, attn_w_nopos, attn_b_nopos, pe, Wout,
           bout):
    del attn_b_pos, attn_b_nopos
    nodes_i = nodes.astype(jnp.int32).reshape(B)
    idx1 = nbr1.astype(jnp.int32).reshape(G1_ROWS)
    idx2 = nbr2.astype(jnp.int32).reshape(H2_ROWS)
    g1a, a1a, h0 = _sc_gather(x, idx1, idx2, nodes_i, 0)
    g1b, a1b = _sc_gather(x, idx1, idx2, nodes_i, 1)
    w1c = jnp.concatenate([W1_self, W1_nbr], axis=0).astype(jnp.bfloat16)
    w2c = jnp.concatenate([W2_self, W2_nbr], axis=0).astype(jnp.bfloat16)
    seq_a = _tc_half1(g1a.reshape(S1, TH, B, D), a1a.reshape(S1, TH, B, D),
                      h0, w1c, w2c)
    return _tc_half2(
        g1b.reshape(S1, TH, B, D), a1b.reshape(S1, TH, B, D), h0, seq_a,
        w1c, w2c, attn_w_pos, attn_w_nopos, pe, Wout, bout.reshape(1, D))
